# Initial kernel scaffold; baseline (speedup 1.0000x reference)
import jax
import jax.numpy as jnp
from jax.experimental import pallas as pl


def kernel(feat, src12, dst12, src23, dst23, src34, dst34, g1, g2, g3, g4, W2, b2, W3, b3, W4, b4, Ws1, bs1, Ws2, bs2):
    raise NotImplementedError("write your pallas kernel here")


# speedup vs baseline: 4.4291x; 4.4291x over previous
import functools

import jax
import jax.numpy as jnp
from jax import lax
from jax.experimental import pallas as pl
from jax.experimental.pallas import tpu as pltpu
from jax.experimental.pallas import tpu_sc as plsc

N = 10000
E = 320000
G = 512
D = 128
NC = 2
NS = 16
NW = NC * NS

CH = 80
EPT = E // NW
NCH = EPT // CH
RCH = N // CH
ZR = 200
ZCH = N // ZR
GPT = G // NS

_mesh = plsc.VectorSubcoreMesh(core_axis_name="c", subcore_axis_name="s")


def _seg_body(h_hbm, src_hbm, dst_hbm, g_hbm, z_hbm,
              m_out, p_out,
              idx_src, idx_dst, idx_g, rows, obuf, gsem,
              acc_m, acc_p):
    cid = lax.axis_index("c")
    sid = lax.axis_index("s")
    wid = sid * NC + cid

    def zero_body(k, _):
        c = sid + k * NS

        @pl.when(c < ZCH)
        def _():
            pltpu.sync_copy(z_hbm, acc_m.at[pl.ds(c * ZR, ZR), :])

        return 0

    lax.fori_loop(0, (ZCH + NS - 1) // NS, zero_body, 0)
    pltpu.sync_copy(z_hbm.at[pl.ds(0, GPT), :], acc_p.at[pl.ds(sid * GPT, GPT), :])
    plsc.subcore_barrier()

    def edge_body(j, _):
        base = wid * EPT + j * CH
        pltpu.sync_copy(src_hbm.at[pl.ds(base, CH)], idx_src)
        pltpu.sync_copy(dst_hbm.at[pl.ds(base, CH)], idx_dst)
        pltpu.async_copy(h_hbm.at[idx_src], rows, gsem).wait()
        pltpu.sync_copy(rows, acc_m.at[idx_dst], add=True)
        return 0

    lax.fori_loop(0, NCH, edge_body, 0)

    def pool_body(k, _):
        c = wid + k * NW

        @pl.when(c < RCH)
        def _():
            pltpu.sync_copy(g_hbm.at[pl.ds(c * CH, CH)], idx_g)
            pltpu.sync_copy(h_hbm.at[pl.ds(c * CH, CH), :], rows)
            pltpu.sync_copy(rows, acc_p.at[idx_g], add=True)

        return 0

    lax.fori_loop(0, (RCH + NW - 1) // NW, pool_body, 0)

    plsc.subcore_barrier()

    def wr_body(k, _):
        c = sid + k * NS

        @pl.when(c < ZCH)
        def _():
            r0 = c * ZR
            pltpu.sync_copy(acc_m.at[pl.ds(r0, ZR), :], obuf)
            pltpu.sync_copy(obuf, m_out.at[cid, pl.ds(r0, ZR), :])

        return 0

    lax.fori_loop(0, (ZCH + NS - 1) // NS, wr_body, 0)
    pltpu.sync_copy(acc_p.at[pl.ds(sid * GPT, GPT), :], obuf.at[pl.ds(0, GPT), :])
    pltpu.sync_copy(obuf.at[pl.ds(0, GPT), :], p_out.at[cid, pl.ds(sid * GPT, GPT), :])


_seg_call = pl.kernel(
    _seg_body,
    out_type=(
        jax.ShapeDtypeStruct((NC, N, D), jnp.float32),
        jax.ShapeDtypeStruct((NC, G, D), jnp.float32),
    ),
    mesh=_mesh,
    scratch_types=[
        pltpu.VMEM((CH,), jnp.int32),
        pltpu.VMEM((CH,), jnp.int32),
        pltpu.VMEM((CH,), jnp.int32),
        pltpu.VMEM((CH, D), jnp.float32),
        pltpu.VMEM((ZR, D), jnp.float32),
        pltpu.SemaphoreType.DMA,
        pltpu.VMEM_SHARED((N, D), jnp.float32),
        pltpu.VMEM_SHARED((G, D), jnp.float32),
    ],
)


def _pool_body(h_hbm, g_hbm, z_hbm, p_out, idx_g, rows, gsem, acc_p):
    cid = lax.axis_index("c")
    sid = lax.axis_index("s")
    wid = sid * NC + cid

    pltpu.sync_copy(z_hbm.at[pl.ds(0, GPT), :], acc_p.at[pl.ds(sid * GPT, GPT), :])
    plsc.subcore_barrier()

    def pool_body(k, _):
        c = wid + k * NW

        @pl.when(c < RCH)
        def _():
            pltpu.sync_copy(g_hbm.at[pl.ds(c * CH, CH)], idx_g)
            pltpu.sync_copy(h_hbm.at[pl.ds(c * CH, CH), :], rows)
            pltpu.sync_copy(rows, acc_p.at[idx_g], add=True)

        return 0

    lax.fori_loop(0, (RCH + NW - 1) // NW, pool_body, 0)
    plsc.subcore_barrier()
    pltpu.sync_copy(acc_p.at[pl.ds(sid * GPT, GPT), :], rows.at[pl.ds(0, GPT), :])
    pltpu.sync_copy(rows.at[pl.ds(0, GPT), :], p_out.at[cid, pl.ds(sid * GPT, GPT), :])


_pool_call = pl.kernel(
    _pool_body,
    out_type=jax.ShapeDtypeStruct((NC, G, D), jnp.float32),
    mesh=_mesh,
    scratch_types=[
        pltpu.VMEM((CH,), jnp.int32),
        pltpu.VMEM((CH, D), jnp.float32),
        pltpu.SemaphoreType.DMA,
        pltpu.VMEM_SHARED((G, D), jnp.float32),
    ],
)


def _silu_tc_body(x_ref, o_ref):
    x = x_ref[...]
    o_ref[...] = x * jax.nn.sigmoid(x)


def _silu_tc(x):
    bn = 2000
    return pl.pallas_call(
        _silu_tc_body,
        out_shape=jax.ShapeDtypeStruct((N, D), jnp.float32),
        grid=(N // bn,),
        in_specs=[pl.BlockSpec((bn, D), lambda i: (i, 0))],
        out_specs=pl.BlockSpec((bn, D), lambda i: (i, 0)),
    )(x)


def _mm_tc_body(m_ref, w_ref, b_ref, o_ref):
    x = m_ref[0] + m_ref[1]
    y = jnp.dot(x, w_ref[...], preferred_element_type=jnp.float32) + b_ref[...]
    o_ref[...] = y * jax.nn.sigmoid(y)


def _mm_tc(m_parts, w, b):
    bn = 2000
    return pl.pallas_call(
        _mm_tc_body,
        out_shape=jax.ShapeDtypeStruct((N, D), jnp.float32),
        grid=(N // bn,),
        in_specs=[
            pl.BlockSpec((NC, bn, D), lambda i: (0, i, 0)),
            pl.BlockSpec((D, D), lambda i: (0, 0)),
            pl.BlockSpec((1, D), lambda i: (0, 0)),
        ],
        out_specs=pl.BlockSpec((bn, D), lambda i: (i, 0)),
    )(m_parts, w, b.reshape(1, D))


def _final_tc_body(p1_ref, p2_ref, p3_ref, p4_ref, ws1_ref, bs1_ref,
                   ws2_ref, bs2_ref, o_ref):
    t = bs1_ref[...]
    for l, ref in enumerate((p1_ref, p2_ref, p3_ref, p4_ref)):
        p = ref[0] + ref[1]
        t = t + jnp.dot(p, ws1_ref[l * D:(l + 1) * D, :],
                        preferred_element_type=jnp.float32)
    u = t * jax.nn.sigmoid(t)
    o_ref[...] = jnp.dot(u, ws2_ref[...],
                         preferred_element_type=jnp.float32) + bs2_ref[...]


def _final_tc(p1, p2, p3, p4, ws1, bs1, ws2, bs2):
    full = lambda s: pl.BlockSpec(s, lambda: tuple(0 for _ in s))
    return pl.pallas_call(
        _final_tc_body,
        out_shape=jax.ShapeDtypeStruct((G, D), jnp.float32),
        in_specs=[
            full((NC, G, D)), full((NC, G, D)), full((NC, G, D)), full((NC, G, D)),
            full((4 * D, D)), full((1, D)), full((D, D)), full((1, D)),
        ],
        out_specs=full((G, D)),
    )(p1, p2, p3, p4, ws1, bs1.reshape(1, D), ws2, bs2.reshape(1, D))


def kernel(feat, src12, dst12, src23, dst23, src34, dst34, g1, g2, g3, g4,
           W2, b2, W3, b3, W4, b4, Ws1, bs1, Ws2, bs2):
    z = jnp.zeros((ZR, D), jnp.float32)
    h1 = _silu_tc(feat)
    m2p, p1p = _seg_call(h1, src12, dst12, g1, z)
    h2 = _mm_tc(m2p, W2, b2)
    m3p, p2p = _seg_call(h2, src23, dst23, g2, z)
    h3 = _mm_tc(m3p, W3, b3)
    m4p, p3p = _seg_call(h3, src34, dst34, g3, z)
    h4 = _mm_tc(m4p, W4, b4)
    p4p = _pool_call(h4, g4, z)
    return _final_tc(p1p, p2p, p3p, p4p, Ws1, bs1, Ws2, bs2)


# speedup vs baseline: 8.8469x; 1.9975x over previous
import functools

import jax
import jax.numpy as jnp
from jax import lax
from jax.experimental import pallas as pl
from jax.experimental.pallas import tpu as pltpu
from jax.experimental.pallas import tpu_sc as plsc

N = 10000
E = 320000
G = 512
D = 128
NC = 2
NS = 16
NW = NC * NS

CH = 80
EPT = E // NW
NCH = EPT // CH
RCH = N // CH
ZR = 80
ZCH = N // ZR
GPT = G // NS

_mesh = plsc.VectorSubcoreMesh(core_axis_name="c", subcore_axis_name="s")
NB = 2
BCH = 25
NBLK = NCH // BCH


def _seg_body(h_hbm, src4_hbm, dst4_hbm, g_hbm, z_hbm,
              m_out, p_out,
              idx_src, idx_dst, idx_g, rows,
              gsem0, gsem1,
              acc_m, acc_p):
    cid = lax.axis_index("c")
    sid = lax.axis_index("s")
    wid = sid * NC + cid
    gsems = (gsem0, gsem1)

    def zero_body(k, _):
        c = sid + k * NS

        @pl.when(c < ZCH)
        def _():
            pltpu.sync_copy(z_hbm, acc_m.at[pl.ds(c * ZR, ZR), :])

        return 0

    lax.fori_loop(0, (ZCH + NS - 1) // NS, zero_body, 0)
    pltpu.sync_copy(z_hbm.at[pl.ds(0, GPT), :], acc_p.at[pl.ds(sid * GPT, GPT), :])
    plsc.subcore_barrier()

    def g_start(j, b):
        pltpu.make_async_copy(h_hbm.at[idx_src.at[j]], rows.at[b], gsems[b]).start()

    def g_wait(j, b):
        pltpu.make_async_copy(h_hbm.at[idx_src.at[j]], rows.at[b], gsems[b]).wait()

    def edge_block(bo, _):
        pltpu.sync_copy(src4_hbm.at[wid, bo], idx_src)
        pltpu.sync_copy(dst4_hbm.at[wid, bo], idx_dst)
        for b in range(NB):
            g_start(b, b)

        def edge_inner(t, _):
            for b in range(NB):
                j = t * NB + b

                def step(jj=j, bb=b):
                    g_wait(jj, bb)
                    pltpu.sync_copy(rows.at[bb], acc_m.at[idx_dst.at[jj]], add=True)

                    @pl.when(jj + NB < BCH)
                    def _():
                        g_start(jj + NB, bb)

                pl.when(j < BCH)(step)
            return 0

        lax.fori_loop(0, (BCH + NB - 1) // NB, edge_inner, 0)
        return 0

    lax.fori_loop(0, NBLK, edge_block, 0)

    def pool_body(k, _):
        c = wid + k * NW

        @pl.when(c < RCH)
        def _():
            pltpu.sync_copy(g_hbm.at[pl.ds(c * CH, CH)], idx_g)
            pltpu.sync_copy(h_hbm.at[pl.ds(c * CH, CH), :], rows.at[0])
            pltpu.sync_copy(rows.at[0], acc_p.at[idx_g], add=True)

        return 0

    lax.fori_loop(0, (RCH + NW - 1) // NW, pool_body, 0)

    plsc.subcore_barrier()

    def wr_body(k, _):
        c = sid + k * NS

        @pl.when(c < ZCH)
        def _():
            r0 = c * ZR
            pltpu.sync_copy(acc_m.at[pl.ds(r0, ZR), :], m_out.at[cid, pl.ds(r0, ZR), :])

        return 0

    lax.fori_loop(0, (ZCH + NS - 1) // NS, wr_body, 0)
    pltpu.sync_copy(acc_p.at[pl.ds(sid * GPT, GPT), :], p_out.at[cid, pl.ds(sid * GPT, GPT), :])


_seg_call = pl.kernel(
    _seg_body,
    out_type=(
        jax.ShapeDtypeStruct((NC, N, D), jnp.float32),
        jax.ShapeDtypeStruct((NC, G, D), jnp.float32),
    ),
    mesh=_mesh,
    scratch_types=[
        pltpu.VMEM((BCH, CH), jnp.int32),
        pltpu.VMEM((BCH, CH), jnp.int32),
        pltpu.VMEM((CH,), jnp.int32),
        pltpu.VMEM((NB, CH, D), jnp.float32),
        pltpu.SemaphoreType.DMA,
        pltpu.SemaphoreType.DMA,
        pltpu.VMEM_SHARED((N, D), jnp.float32),
        pltpu.VMEM_SHARED((G, D), jnp.float32),
    ],
)


def _pool_body(h_hbm, g_hbm, z_hbm, p_out, idx_g, rows, acc_p):
    cid = lax.axis_index("c")
    sid = lax.axis_index("s")
    wid = sid * NC + cid

    pltpu.sync_copy(z_hbm.at[pl.ds(0, GPT), :], acc_p.at[pl.ds(sid * GPT, GPT), :])
    plsc.subcore_barrier()

    def pool_body(k, _):
        c = wid + k * NW

        @pl.when(c < RCH)
        def _():
            pltpu.sync_copy(g_hbm.at[pl.ds(c * CH, CH)], idx_g)
            pltpu.sync_copy(h_hbm.at[pl.ds(c * CH, CH), :], rows)
            pltpu.sync_copy(rows, acc_p.at[idx_g], add=True)

        return 0

    lax.fori_loop(0, (RCH + NW - 1) // NW, pool_body, 0)
    plsc.subcore_barrier()
    pltpu.sync_copy(acc_p.at[pl.ds(sid * GPT, GPT), :], p_out.at[cid, pl.ds(sid * GPT, GPT), :])


_pool_call = pl.kernel(
    _pool_body,
    out_type=jax.ShapeDtypeStruct((NC, G, D), jnp.float32),
    mesh=_mesh,
    scratch_types=[
        pltpu.VMEM((CH,), jnp.int32),
        pltpu.VMEM((CH, D), jnp.float32),
        pltpu.VMEM_SHARED((G, D), jnp.float32),
    ],
)


def _silu_tc_body(x_ref, o_ref):
    x = x_ref[...]
    o_ref[...] = x * jax.nn.sigmoid(x)


def _silu_tc(x):
    bn = 2000
    return pl.pallas_call(
        _silu_tc_body,
        out_shape=jax.ShapeDtypeStruct((N, D), jnp.float32),
        grid=(N // bn,),
        in_specs=[pl.BlockSpec((bn, D), lambda i: (i, 0))],
        out_specs=pl.BlockSpec((bn, D), lambda i: (i, 0)),
    )(x)


def _mm_tc_body(m_ref, w_ref, b_ref, o_ref):
    x = m_ref[0] + m_ref[1]
    y = jnp.dot(x, w_ref[...], preferred_element_type=jnp.float32) + b_ref[...]
    o_ref[...] = y * jax.nn.sigmoid(y)


def _mm_tc(m_parts, w, b):
    bn = 2000
    return pl.pallas_call(
        _mm_tc_body,
        out_shape=jax.ShapeDtypeStruct((N, D), jnp.float32),
        grid=(N // bn,),
        in_specs=[
            pl.BlockSpec((NC, bn, D), lambda i: (0, i, 0)),
            pl.BlockSpec((D, D), lambda i: (0, 0)),
            pl.BlockSpec((1, D), lambda i: (0, 0)),
        ],
        out_specs=pl.BlockSpec((bn, D), lambda i: (i, 0)),
    )(m_parts, w, b.reshape(1, D))


def _final_tc_body(p1_ref, p2_ref, p3_ref, p4_ref, ws1_ref, bs1_ref,
                   ws2_ref, bs2_ref, o_ref):
    t = bs1_ref[...]
    for l, ref in enumerate((p1_ref, p2_ref, p3_ref, p4_ref)):
        p = ref[0] + ref[1]
        t = t + jnp.dot(p, ws1_ref[l * D:(l + 1) * D, :],
                        preferred_element_type=jnp.float32)
    u = t * jax.nn.sigmoid(t)
    o_ref[...] = jnp.dot(u, ws2_ref[...],
                         preferred_element_type=jnp.float32) + bs2_ref[...]


def _final_tc(p1, p2, p3, p4, ws1, bs1, ws2, bs2):
    full = lambda s: pl.BlockSpec(s, lambda: tuple(0 for _ in s))
    return pl.pallas_call(
        _final_tc_body,
        out_shape=jax.ShapeDtypeStruct((G, D), jnp.float32),
        in_specs=[
            full((NC, G, D)), full((NC, G, D)), full((NC, G, D)), full((NC, G, D)),
            full((4 * D, D)), full((1, D)), full((D, D)), full((1, D)),
        ],
        out_specs=full((G, D)),
    )(p1, p2, p3, p4, ws1, bs1.reshape(1, D), ws2, bs2.reshape(1, D))


def kernel(feat, src12, dst12, src23, dst23, src34, dst34, g1, g2, g3, g4,
           W2, b2, W3, b3, W4, b4, Ws1, bs1, Ws2, bs2):
    z = jnp.zeros((ZR, D), jnp.float32)
    r3 = lambda a: a.reshape(NW, NBLK, BCH, CH)
    h1 = _silu_tc(feat)
    m2p, p1p = _seg_call(h1, r3(src12), r3(dst12), g1, z)
    h2 = _mm_tc(m2p, W2, b2)
    m3p, p2p = _seg_call(h2, r3(src23), r3(dst23), g2, z)
    h3 = _mm_tc(m3p, W3, b3)
    m4p, p3p = _seg_call(h3, r3(src34), r3(dst34), g3, z)
    h4 = _mm_tc(m4p, W4, b4)
    p4p = _pool_call(h4, g4, z)
    return _final_tc(p1p, p2p, p3p, p4p, Ws1, bs1, Ws2, bs2)
